# baseline (speedup 1.0000x reference)
import functools

import jax
import jax.numpy as jnp
from jax.experimental import pallas as pl
from jax.experimental.pallas import tpu as pltpu

_EPS = 1e-12


def _prep_body(fw_ref, w1_ref, mn_ref, vp_ref, *, c, hdim):
    fw = fw_ref[...]
    nf = jnp.sqrt(jnp.sum(fw * fw, axis=1, keepdims=True))
    mn_full = fw / jnp.maximum(nf, _EPS)
    vp_ref[:, :hdim] = jnp.dot(mn_full, w1_ref[...],
                               preferred_element_type=jnp.float32
                               ).astype(jnp.bfloat16)
    vp_ref[:, hdim:] = jnp.ones_like(vp_ref[:, hdim:])
    m = fw[:, :c]
    nm = jnp.sqrt(jnp.sum(m * m, axis=1, keepdims=True))
    mn_ref[...] = (m / jnp.maximum(nm, _EPS)).astype(jnp.bfloat16)


def _qk(q, mn_ref, t, ck):
    return jax.lax.dot_general(q, mn_ref[pl.ds(t * ck, ck), :],
                               (((1,), (1,)), ((), ())),
                               preferred_element_type=jnp.float32)


def _flash_body(x_ref, mn_ref, vp_ref, b1_ref, w2_ref, b2_ref, o_ref,
                p_scr, *, hdim, ck, nk):
    xq = x_ref[...]
    nq = jnp.sqrt(jnp.sum(xq * xq, axis=1, keepdims=True))
    q = (xq / jnp.maximum(nq, _EPS)).astype(jnp.bfloat16)

    def body(t, _):
        s = _qk(q, mn_ref, t, ck)
        p_scr[:, pl.ds(t * ck, ck)] = jnp.exp(s).astype(jnp.bfloat16)
        return 0

    jax.lax.fori_loop(0, nk, body, 0)

    acc = jnp.dot(p_scr[...], vp_ref[...], preferred_element_type=jnp.float32)

    z = acc[:, :hdim] / acc[:, hdim:hdim + 1] + b1_ref[...]
    h1 = 0.5 * z * (1.0 + jax.lax.erf(z * (2.0 ** -0.5)))
    o_ref[...] = jnp.dot(h1.astype(jnp.bfloat16), w2_ref[...],
                         preferred_element_type=jnp.float32) + b2_ref[...]


def kernel(x, feat_w, W1, b1, W2, b2):
    b, c, h, w = x.shape
    n = b * h * w
    kdim, cf = feat_w.shape
    hdim = W1.shape[1]
    x_flat = jnp.transpose(x, (0, 2, 3, 1)).reshape(n, c)

    hext = hdim + 128
    BKP = 1024
    mn, vp = pl.pallas_call(
        functools.partial(_prep_body, c=c, hdim=hdim),
        grid=(kdim // BKP,),
        in_specs=[pl.BlockSpec((BKP, cf), lambda i: (i, 0)),
                  pl.BlockSpec((cf, hdim), lambda i: (0, 0))],
        out_specs=[pl.BlockSpec((BKP, c), lambda i: (i, 0)),
                   pl.BlockSpec((BKP, hext), lambda i: (i, 0))],
        out_shape=[jax.ShapeDtypeStruct((kdim, c), jnp.bfloat16),
                   jax.ShapeDtypeStruct((kdim, hext), jnp.bfloat16)],
    )(feat_w, W1)

    BQ, CK = 1024, 512
    out = pl.pallas_call(
        functools.partial(_flash_body, hdim=hdim, ck=CK, nk=kdim // CK),
        grid=(n // BQ,),
        in_specs=[pl.BlockSpec((BQ, c), lambda i: (i, 0)),
                  pl.BlockSpec((kdim, c), lambda i: (0, 0)),
                  pl.BlockSpec((kdim, hext), lambda i: (0, 0)),
                  pl.BlockSpec((1, hdim), lambda i: (0, 0)),
                  pl.BlockSpec((hdim, hdim), lambda i: (0, 0)),
                  pl.BlockSpec((1, hdim), lambda i: (0, 0))],
        out_specs=pl.BlockSpec((BQ, hdim), lambda i: (i, 0)),
        out_shape=jax.ShapeDtypeStruct((n, hdim), jnp.float32),
        scratch_shapes=[pltpu.VMEM((BQ, kdim), jnp.bfloat16)],
        compiler_params=pltpu.CompilerParams(
            dimension_semantics=("arbitrary",)),
    )(x_flat, mn, vp, b1.reshape(1, hdim), W2.astype(jnp.bfloat16),
      b2.reshape(1, hdim))

    return jnp.transpose(out.reshape(b, h, w, hdim), (0, 3, 1, 2))


# speedup vs baseline: 1.1352x; 1.1352x over previous
import functools

import jax
import jax.numpy as jnp
from jax.experimental import pallas as pl
from jax.experimental.pallas import tpu as pltpu

_EPS = 1e-12


def _prep_body(fw_ref, w1_ref, mn_ref, vp_ref, *, c, hdim):
    fw = fw_ref[...]
    nf = jnp.sqrt(jnp.sum(fw * fw, axis=1, keepdims=True))
    mn_full = fw / jnp.maximum(nf, _EPS)
    vp_ref[:, :hdim] = jnp.dot(mn_full, w1_ref[...],
                               preferred_element_type=jnp.float32
                               ).astype(jnp.bfloat16)
    vp_ref[:, hdim:] = jnp.ones_like(vp_ref[:, hdim:])
    m = fw[:, :c]
    nm = jnp.sqrt(jnp.sum(m * m, axis=1, keepdims=True))
    mn_ref[...] = (m / jnp.maximum(nm, _EPS)).astype(jnp.bfloat16)


def _qk(q, mn_ref, t, ck):
    return jax.lax.dot_general(q, mn_ref[pl.ds(t * ck, ck), :],
                               (((1,), (1,)), ((), ())),
                               preferred_element_type=jnp.float32)


def _flash_body(x_ref, mn_ref, vp_ref, b1_ref, w2_ref, b2_ref, o_ref,
                p_scr, *, hdim, ck, nk):
    xq = x_ref[...]
    nq = jnp.sqrt(jnp.sum(xq * xq, axis=1, keepdims=True))
    q = (xq / jnp.maximum(nq, _EPS)).astype(jnp.bfloat16)

    def body(t, _):
        s = _qk(q, mn_ref, t, ck)
        p_scr[:, pl.ds(t * ck, ck)] = jnp.exp(s).astype(jnp.bfloat16)
        return 0

    jax.lax.fori_loop(0, nk, body, 0)

    acc = jnp.dot(p_scr[...], vp_ref[...], preferred_element_type=jnp.float32)

    z = acc[:, :hdim] / acc[:, hdim:hdim + 1] + b1_ref[...]
    h1 = 0.5 * z * (1.0 + jax.lax.erf(z * (2.0 ** -0.5)))
    o_ref[...] = jnp.dot(h1.astype(jnp.bfloat16), w2_ref[...],
                         preferred_element_type=jnp.float32) + b2_ref[...]


def kernel(x, feat_w, W1, b1, W2, b2):
    b, c, h, w = x.shape
    n = b * h * w
    kdim, cf = feat_w.shape
    hdim = W1.shape[1]
    x_flat = jnp.transpose(x, (0, 2, 3, 1)).reshape(n, c)

    hext = hdim + 128
    BKP = 1024
    mn, vp = pl.pallas_call(
        functools.partial(_prep_body, c=c, hdim=hdim),
        grid=(kdim // BKP,),
        in_specs=[pl.BlockSpec((BKP, cf), lambda i: (i, 0)),
                  pl.BlockSpec((cf, hdim), lambda i: (0, 0))],
        out_specs=[pl.BlockSpec((BKP, c), lambda i: (i, 0)),
                   pl.BlockSpec((BKP, hext), lambda i: (i, 0))],
        out_shape=[jax.ShapeDtypeStruct((kdim, c), jnp.bfloat16),
                   jax.ShapeDtypeStruct((kdim, hext), jnp.bfloat16)],
    )(feat_w, W1)

    BQ, CK = 1024, 1024
    out = pl.pallas_call(
        functools.partial(_flash_body, hdim=hdim, ck=CK, nk=kdim // CK),
        grid=(n // BQ,),
        in_specs=[pl.BlockSpec((BQ, c), lambda i: (i, 0)),
                  pl.BlockSpec((kdim, c), lambda i: (0, 0)),
                  pl.BlockSpec((kdim, hext), lambda i: (0, 0)),
                  pl.BlockSpec((1, hdim), lambda i: (0, 0)),
                  pl.BlockSpec((hdim, hdim), lambda i: (0, 0)),
                  pl.BlockSpec((1, hdim), lambda i: (0, 0))],
        out_specs=pl.BlockSpec((BQ, hdim), lambda i: (i, 0)),
        out_shape=jax.ShapeDtypeStruct((n, hdim), jnp.float32),
        scratch_shapes=[pltpu.VMEM((BQ, kdim), jnp.bfloat16)],
        compiler_params=pltpu.CompilerParams(
            dimension_semantics=("arbitrary",)),
    )(x_flat, mn, vp, b1.reshape(1, hdim), W2.astype(jnp.bfloat16),
      b2.reshape(1, hdim))

    return jnp.transpose(out.reshape(b, h, w, hdim), (0, 3, 1, 2))


# speedup vs baseline: 1.2084x; 1.0645x over previous
import functools

import jax
import jax.numpy as jnp
from jax.experimental import pallas as pl
from jax.experimental.pallas import tpu as pltpu

_EPS = 1e-12


def _prep_body(fw_ref, w1_ref, mn_ref, vp_ref, *, c, hdim):
    fw = fw_ref[...]
    nf = jnp.sqrt(jnp.sum(fw * fw, axis=1, keepdims=True))
    mn_full = fw / jnp.maximum(nf, _EPS)
    vp_ref[:, :hdim] = jnp.dot(mn_full, w1_ref[...],
                               preferred_element_type=jnp.float32
                               ).astype(jnp.bfloat16)
    vp_ref[:, hdim:] = jnp.ones_like(vp_ref[:, hdim:])
    m = fw[:, :c]
    nm = jnp.sqrt(jnp.sum(m * m, axis=1, keepdims=True))
    mn_ref[...] = (m / jnp.maximum(nm, _EPS)).astype(jnp.bfloat16)


def _qk(q, mn_ref, t, ck):
    return jax.lax.dot_general(q, mn_ref[pl.ds(t * ck, ck), :],
                               (((1,), (1,)), ((), ())),
                               preferred_element_type=jnp.float32)


def _flash_body(x_ref, mn_ref, vp_ref, b1_ref, w2_ref, b2_ref, o_ref,
                p_scr, *, hdim, ck, nk):
    xq = x_ref[...]
    nq = jnp.sqrt(jnp.sum(xq * xq, axis=1, keepdims=True))
    q = (xq / jnp.maximum(nq, _EPS)).astype(jnp.bfloat16)

    def body(t, _):
        s = _qk(q, mn_ref, t, ck)
        p_scr[:, pl.ds(t * ck, ck)] = jnp.exp(s).astype(jnp.bfloat16)
        return 0

    jax.lax.fori_loop(0, nk, body, 0)

    acc = jnp.dot(p_scr[...], vp_ref[...], preferred_element_type=jnp.float32)

    z = acc[:, :hdim] / acc[:, hdim:hdim + 1] + b1_ref[...]
    h1 = 0.5 * z * (1.0 + jax.lax.erf(z * (2.0 ** -0.5)))
    o_ref[...] = jnp.dot(h1.astype(jnp.bfloat16), w2_ref[...],
                         preferred_element_type=jnp.float32) + b2_ref[...]


def kernel(x, feat_w, W1, b1, W2, b2):
    b, c, h, w = x.shape
    n = b * h * w
    kdim, cf = feat_w.shape
    hdim = W1.shape[1]
    x_flat = jnp.transpose(x, (0, 2, 3, 1)).reshape(n, c)

    hext = hdim + 128
    BKP = 1024
    mn, vp = pl.pallas_call(
        functools.partial(_prep_body, c=c, hdim=hdim),
        grid=(kdim // BKP,),
        in_specs=[pl.BlockSpec((BKP, cf), lambda i: (i, 0)),
                  pl.BlockSpec((cf, hdim), lambda i: (0, 0))],
        out_specs=[pl.BlockSpec((BKP, c), lambda i: (i, 0)),
                   pl.BlockSpec((BKP, hext), lambda i: (i, 0))],
        out_shape=[jax.ShapeDtypeStruct((kdim, c), jnp.bfloat16),
                   jax.ShapeDtypeStruct((kdim, hext), jnp.bfloat16)],
    )(feat_w, W1)

    BQ, CK = 1024, 2048
    out = pl.pallas_call(
        functools.partial(_flash_body, hdim=hdim, ck=CK, nk=kdim // CK),
        grid=(n // BQ,),
        in_specs=[pl.BlockSpec((BQ, c), lambda i: (i, 0)),
                  pl.BlockSpec((kdim, c), lambda i: (0, 0)),
                  pl.BlockSpec((kdim, hext), lambda i: (0, 0)),
                  pl.BlockSpec((1, hdim), lambda i: (0, 0)),
                  pl.BlockSpec((hdim, hdim), lambda i: (0, 0)),
                  pl.BlockSpec((1, hdim), lambda i: (0, 0))],
        out_specs=pl.BlockSpec((BQ, hdim), lambda i: (i, 0)),
        out_shape=jax.ShapeDtypeStruct((n, hdim), jnp.float32),
        scratch_shapes=[pltpu.VMEM((BQ, kdim), jnp.bfloat16)],
        compiler_params=pltpu.CompilerParams(
            dimension_semantics=("arbitrary",)),
    )(x_flat, mn, vp, b1.reshape(1, hdim), W2.astype(jnp.bfloat16),
      b2.reshape(1, hdim))

    return jnp.transpose(out.reshape(b, h, w, hdim), (0, 3, 1, 2))


# speedup vs baseline: 1.2519x; 1.0359x over previous
import functools

import jax
import jax.numpy as jnp
from jax.experimental import pallas as pl
from jax.experimental.pallas import tpu as pltpu

_EPS = 1e-12


def _prep_body(fw_ref, w1_ref, mn_ref, vp_ref, *, c, hdim):
    fw = fw_ref[...]
    nf = jnp.sqrt(jnp.sum(fw * fw, axis=1, keepdims=True))
    mn_full = fw / jnp.maximum(nf, _EPS)
    vp_ref[:, :hdim] = jnp.dot(mn_full, w1_ref[...],
                               preferred_element_type=jnp.float32
                               ).astype(jnp.bfloat16)
    vp_ref[:, hdim:] = jnp.ones_like(vp_ref[:, hdim:])
    m = fw[:, :c]
    nm = jnp.sqrt(jnp.sum(m * m, axis=1, keepdims=True))
    mn_ref[...] = (m / jnp.maximum(nm, _EPS)).astype(jnp.bfloat16)


def _qk(q, mn_ref, t, ck):
    return jax.lax.dot_general(q, mn_ref[pl.ds(t * ck, ck), :],
                               (((1,), (1,)), ((), ())),
                               preferred_element_type=jnp.float32)


def _flash_body(x_ref, mn_ref, vp_ref, b1_ref, w2_ref, b2_ref, o_ref,
                p_scr, *, hdim, ck, nk):
    xq = x_ref[...]
    nq = jnp.sqrt(jnp.sum(xq * xq, axis=1, keepdims=True))
    q = (xq / jnp.maximum(nq, _EPS)).astype(jnp.bfloat16)

    def body(t, _):
        s = _qk(q, mn_ref, t, ck)
        p_scr[:, pl.ds(t * ck, ck)] = jnp.exp(s).astype(jnp.bfloat16)
        return 0

    jax.lax.fori_loop(0, nk, body, 0)

    acc = jnp.dot(p_scr[...], vp_ref[...], preferred_element_type=jnp.float32)

    z = acc[:, :hdim] / acc[:, hdim:hdim + 1] + b1_ref[...]
    h1 = 0.5 * z * (1.0 + jax.lax.erf(z * (2.0 ** -0.5)))
    o_ref[...] = jnp.dot(h1.astype(jnp.bfloat16), w2_ref[...],
                         preferred_element_type=jnp.float32) + b2_ref[...]


def kernel(x, feat_w, W1, b1, W2, b2):
    b, c, h, w = x.shape
    n = b * h * w
    kdim, cf = feat_w.shape
    hdim = W1.shape[1]
    x_flat = jnp.transpose(x, (0, 2, 3, 1)).reshape(n, c)

    hext = hdim + 128
    BKP = 1024
    mn, vp = pl.pallas_call(
        functools.partial(_prep_body, c=c, hdim=hdim),
        grid=(kdim // BKP,),
        in_specs=[pl.BlockSpec((BKP, cf), lambda i: (i, 0)),
                  pl.BlockSpec((cf, hdim), lambda i: (0, 0))],
        out_specs=[pl.BlockSpec((BKP, c), lambda i: (i, 0)),
                   pl.BlockSpec((BKP, hext), lambda i: (i, 0))],
        out_shape=[jax.ShapeDtypeStruct((kdim, c), jnp.bfloat16),
                   jax.ShapeDtypeStruct((kdim, hext), jnp.bfloat16)],
    )(feat_w, W1)

    BQ, CK = 2048, 2048
    out = pl.pallas_call(
        functools.partial(_flash_body, hdim=hdim, ck=CK, nk=kdim // CK),
        grid=(n // BQ,),
        in_specs=[pl.BlockSpec((BQ, c), lambda i: (i, 0)),
                  pl.BlockSpec((kdim, c), lambda i: (0, 0)),
                  pl.BlockSpec((kdim, hext), lambda i: (0, 0)),
                  pl.BlockSpec((1, hdim), lambda i: (0, 0)),
                  pl.BlockSpec((hdim, hdim), lambda i: (0, 0)),
                  pl.BlockSpec((1, hdim), lambda i: (0, 0))],
        out_specs=pl.BlockSpec((BQ, hdim), lambda i: (i, 0)),
        out_shape=jax.ShapeDtypeStruct((n, hdim), jnp.float32),
        scratch_shapes=[pltpu.VMEM((BQ, kdim), jnp.bfloat16)],
        compiler_params=pltpu.CompilerParams(
            dimension_semantics=("arbitrary",)),
    )(x_flat, mn, vp, b1.reshape(1, hdim), W2.astype(jnp.bfloat16),
      b2.reshape(1, hdim))

    return jnp.transpose(out.reshape(b, h, w, hdim), (0, 3, 1, 2))


# speedup vs baseline: 1.2750x; 1.0185x over previous
import functools

import jax
import jax.numpy as jnp
from jax.experimental import pallas as pl
from jax.experimental.pallas import tpu as pltpu

_EPS = 1e-12


def _prep_body(fw_ref, w1_ref, mn_ref, vp_ref, *, c, hdim):
    fw = fw_ref[...]
    nf = jnp.sqrt(jnp.sum(fw * fw, axis=1, keepdims=True))
    mn_full = fw / jnp.maximum(nf, _EPS)
    vp_ref[:, :hdim] = jnp.dot(mn_full, w1_ref[...],
                               preferred_element_type=jnp.float32
                               ).astype(jnp.bfloat16)
    vp_ref[:, hdim:] = jnp.ones_like(vp_ref[:, hdim:])
    m = fw[:, :c]
    nm = jnp.sqrt(jnp.sum(m * m, axis=1, keepdims=True))
    mn_ref[...] = (m / jnp.maximum(nm, _EPS)).astype(jnp.bfloat16)


def _qk(q, mn_ref, t, ck):
    return jax.lax.dot_general(q, mn_ref[pl.ds(t * ck, ck), :],
                               (((1,), (1,)), ((), ())),
                               preferred_element_type=jnp.float32)


def _flash_body(x_ref, mn_ref, vp_ref, b1_ref, w2_ref, b2_ref, o_ref,
                p_scr, *, hdim, ck, nk):
    xq = x_ref[...]
    nq = jnp.sqrt(jnp.sum(xq * xq, axis=1, keepdims=True))
    q = (xq / jnp.maximum(nq, _EPS)).astype(jnp.bfloat16)

    def body(t, _):
        s = _qk(q, mn_ref, t, ck)
        p_scr[:, pl.ds(t * ck, ck)] = jnp.exp(s).astype(jnp.bfloat16)
        return 0

    jax.lax.fori_loop(0, nk, body, 0)

    acc = jnp.dot(p_scr[...], vp_ref[...], preferred_element_type=jnp.float32)

    z = acc[:, :hdim] / acc[:, hdim:hdim + 1] + b1_ref[...]
    h1 = 0.5 * z * (1.0 + jax.lax.erf(z * (2.0 ** -0.5)))
    o_ref[...] = jnp.dot(h1.astype(jnp.bfloat16), w2_ref[...],
                         preferred_element_type=jnp.float32) + b2_ref[...]


def kernel(x, feat_w, W1, b1, W2, b2):
    b, c, h, w = x.shape
    n = b * h * w
    kdim, cf = feat_w.shape
    hdim = W1.shape[1]
    x_flat = jnp.transpose(x, (0, 2, 3, 1)).reshape(n, c)

    hext = hdim + 128
    BKP = 1024
    mn, vp = pl.pallas_call(
        functools.partial(_prep_body, c=c, hdim=hdim),
        grid=(kdim // BKP,),
        in_specs=[pl.BlockSpec((BKP, cf), lambda i: (i, 0)),
                  pl.BlockSpec((cf, hdim), lambda i: (0, 0))],
        out_specs=[pl.BlockSpec((BKP, c), lambda i: (i, 0)),
                   pl.BlockSpec((BKP, hext), lambda i: (i, 0))],
        out_shape=[jax.ShapeDtypeStruct((kdim, c), jnp.bfloat16),
                   jax.ShapeDtypeStruct((kdim, hext), jnp.bfloat16)],
    )(feat_w, W1)

    BQ, CK = 2048, 4096
    out = pl.pallas_call(
        functools.partial(_flash_body, hdim=hdim, ck=CK, nk=kdim // CK),
        grid=(n // BQ,),
        in_specs=[pl.BlockSpec((BQ, c), lambda i: (i, 0)),
                  pl.BlockSpec((kdim, c), lambda i: (0, 0)),
                  pl.BlockSpec((kdim, hext), lambda i: (0, 0)),
                  pl.BlockSpec((1, hdim), lambda i: (0, 0)),
                  pl.BlockSpec((hdim, hdim), lambda i: (0, 0)),
                  pl.BlockSpec((1, hdim), lambda i: (0, 0))],
        out_specs=pl.BlockSpec((BQ, hdim), lambda i: (i, 0)),
        out_shape=jax.ShapeDtypeStruct((n, hdim), jnp.float32),
        scratch_shapes=[pltpu.VMEM((BQ, kdim), jnp.bfloat16)],
        compiler_params=pltpu.CompilerParams(
            dimension_semantics=("arbitrary",)),
    )(x_flat, mn, vp, b1.reshape(1, hdim), W2.astype(jnp.bfloat16),
      b2.reshape(1, hdim))

    return jnp.transpose(out.reshape(b, h, w, hdim), (0, 3, 1, 2))


# speedup vs baseline: 1.3048x; 1.0234x over previous
import functools

import jax
import jax.numpy as jnp
from jax.experimental import pallas as pl
from jax.experimental.pallas import tpu as pltpu

_EPS = 1e-12


def _prep_body(fw_ref, w1_ref, mn_ref, vp_ref, *, c, hdim):
    fw = fw_ref[...]
    nf = jnp.sqrt(jnp.sum(fw * fw, axis=1, keepdims=True))
    mn_full = fw / jnp.maximum(nf, _EPS)
    vp_ref[:, :hdim] = jnp.dot(mn_full, w1_ref[...],
                               preferred_element_type=jnp.float32
                               ).astype(jnp.bfloat16)
    vp_ref[:, hdim:] = jnp.ones_like(vp_ref[:, hdim:])
    m = fw[:, :c]
    nm = jnp.sqrt(jnp.sum(m * m, axis=1, keepdims=True))
    mn_ref[...] = (m / jnp.maximum(nm, _EPS)).astype(jnp.bfloat16)


def _qk(q, mn_ref, t, ck):
    return jax.lax.dot_general(q, mn_ref[pl.ds(t * ck, ck), :],
                               (((1,), (1,)), ((), ())),
                               preferred_element_type=jnp.float32)


def _flash_body(x_ref, mn_ref, vp_ref, b1_ref, w2_ref, b2_ref, o_ref,
                *, hdim, ck, nk):
    xq = x_ref[...]
    nq = jnp.sqrt(jnp.sum(xq * xq, axis=1, keepdims=True))
    q = (xq / jnp.maximum(nq, _EPS)).astype(jnp.bfloat16)

    def pv(p16, t):
        return jnp.dot(p16, vp_ref[pl.ds(t * ck, ck), :],
                       preferred_element_type=jnp.float32)

    s_prev = _qk(q, mn_ref, 0, ck)
    acc = None
    for t in range(1, nk):
        s_cur = _qk(q, mn_ref, t, ck)
        p16 = jnp.exp(s_prev).astype(jnp.bfloat16)
        d = pv(p16, t - 1)
        acc = d if acc is None else acc + d
        s_prev = s_cur
    p16 = jnp.exp(s_prev).astype(jnp.bfloat16)
    acc = acc + pv(p16, nk - 1)

    z = acc[:, :hdim] / acc[:, hdim:hdim + 1] + b1_ref[...]
    h1 = 0.5 * z * (1.0 + jax.lax.erf(z * (2.0 ** -0.5)))
    o_ref[...] = jnp.dot(h1.astype(jnp.bfloat16), w2_ref[...],
                         preferred_element_type=jnp.float32) + b2_ref[...]


def kernel(x, feat_w, W1, b1, W2, b2):
    b, c, h, w = x.shape
    n = b * h * w
    kdim, cf = feat_w.shape
    hdim = W1.shape[1]
    x_flat = jnp.transpose(x, (0, 2, 3, 1)).reshape(n, c)

    hext = hdim + 128
    BKP = 1024
    mn, vp = pl.pallas_call(
        functools.partial(_prep_body, c=c, hdim=hdim),
        grid=(kdim // BKP,),
        in_specs=[pl.BlockSpec((BKP, cf), lambda i: (i, 0)),
                  pl.BlockSpec((cf, hdim), lambda i: (0, 0))],
        out_specs=[pl.BlockSpec((BKP, c), lambda i: (i, 0)),
                   pl.BlockSpec((BKP, hext), lambda i: (i, 0))],
        out_shape=[jax.ShapeDtypeStruct((kdim, c), jnp.bfloat16),
                   jax.ShapeDtypeStruct((kdim, hext), jnp.bfloat16)],
    )(feat_w, W1)

    BQ, CK = 2048, 1024
    out = pl.pallas_call(
        functools.partial(_flash_body, hdim=hdim, ck=CK, nk=kdim // CK),
        grid=(n // BQ,),
        in_specs=[pl.BlockSpec((BQ, c), lambda i: (i, 0)),
                  pl.BlockSpec((kdim, c), lambda i: (0, 0)),
                  pl.BlockSpec((kdim, hext), lambda i: (0, 0)),
                  pl.BlockSpec((1, hdim), lambda i: (0, 0)),
                  pl.BlockSpec((hdim, hdim), lambda i: (0, 0)),
                  pl.BlockSpec((1, hdim), lambda i: (0, 0))],
        out_specs=pl.BlockSpec((BQ, hdim), lambda i: (i, 0)),
        out_shape=jax.ShapeDtypeStruct((n, hdim), jnp.float32),
        compiler_params=pltpu.CompilerParams(
            dimension_semantics=("arbitrary",)),
    )(x_flat, mn, vp, b1.reshape(1, hdim), W2.astype(jnp.bfloat16),
      b2.reshape(1, hdim))

    return jnp.transpose(out.reshape(b, h, w, hdim), (0, 3, 1, 2))
